# baseline (speedup 1.0000x reference)
import functools

import jax
import jax.numpy as jnp
from jax import lax
from jax.experimental import pallas as pl
from jax.experimental.pallas import tpu as pltpu
from jax.experimental.pallas import tpu_sc as plsc

O = 10000
T = 320000
D = 128
NEG = 0.01

NC = 2
NS = 16
GK = 80
EPW = T // (NC * NS)
GCHUNK = EPW // GK
SK = 64
NCHUNK = T // SK
SPS = NCHUNK // NS
SREM = NCHUNK - SPS * NS
CK = 128
CCHUNK = T // CK
OP = 10240
OSLICE = OP // NS


def _lrelu(x):
    return jnp.where(x >= 0, x, NEG * x)


def _premul_body(obj_ref, w1s_ref, w1o_ref, a_ref, b_ref):
    blk = obj_ref[...]
    a_ref[...] = jnp.dot(blk, w1s_ref[...], preferred_element_type=jnp.float32)
    b_ref[...] = jnp.dot(blk, w1o_ref[...], preferred_element_type=jnp.float32)


def _premul(obj_vecs, w1s, w1o):
    bo = 1000
    grid = (O // bo,)
    return pl.pallas_call(
        _premul_body,
        grid=grid,
        in_specs=[
            pl.BlockSpec((bo, D), lambda i: (i, 0)),
            pl.BlockSpec((D, D), lambda i: (0, 0)),
            pl.BlockSpec((D, D), lambda i: (0, 0)),
        ],
        out_specs=[
            pl.BlockSpec((bo, D), lambda i: (i, 0)),
            pl.BlockSpec((bo, D), lambda i: (i, 0)),
        ],
        out_shape=[
            jax.ShapeDtypeStruct((O, D), jnp.float32),
            jax.ShapeDtypeStruct((O, D), jnp.float32),
        ],
    )(obj_vecs, w1s, w1o)


def _gather_body(a_hbm, b_hbm, sidx_hbm, oidx_hbm, g_hbm,
                 sidx_v, oidx_v,
                 buf_a0, buf_a1, buf_b0, buf_b1,
                 sem_a0, sem_a1, sem_b0, sem_b1, sem_w0, sem_w1):
    cid = lax.axis_index("c")
    sid = lax.axis_index("s")
    wid = sid * NC + cid
    base = wid * EPW
    buf_a = (buf_a0, buf_a1)
    buf_b = (buf_b0, buf_b1)
    sem_a = (sem_a0, sem_a1)
    sem_b = (sem_b0, sem_b1)
    sem_w = (sem_w0, sem_w1)
    pltpu.sync_copy(sidx_hbm.at[wid], sidx_v)
    pltpu.sync_copy(oidx_hbm.at[wid], oidx_v)

    def issue(i, s):
        pltpu.async_copy(a_hbm.at[sidx_v.at[i]], buf_a[s], sem_a[s])
        pltpu.async_copy(b_hbm.at[oidx_v.at[i]], buf_b[s], sem_b[s])

    def g_dst(i):
        return g_hbm.at[pl.ds(base + i * GK, GK)]

    issue(0, 0)

    def process(i, s):
        @pl.when(i < GCHUNK)
        def _():
            pltpu.make_async_copy(a_hbm.at[sidx_v.at[i]], buf_a[s], sem_a[s]).wait()
            pltpu.make_async_copy(b_hbm.at[oidx_v.at[i]], buf_b[s], sem_b[s]).wait()

            @pl.when(i >= 1)
            def _():
                pltpu.make_async_copy(buf_a[1 - s], g_dst(i - 1), sem_w[1 - s]).wait()

            @pl.when(i + 1 < GCHUNK)
            def _():
                issue(i + 1, 1 - s)

            @pl.loop(0, GK)
            def _row(r):
                for c in range(D // 16):
                    sl = pl.ds(c * 16, 16)
                    buf_a[s][r, sl] = buf_a[s][r, sl] + buf_b[s][r, sl]

            pltpu.async_copy(buf_a[s], g_dst(i), sem_w[s])

    @pl.loop(0, (GCHUNK + 1) // 2)
    def _outer(k):
        process(2 * k, 0)
        process(2 * k + 1, 1)

    last = GCHUNK - 1
    pltpu.make_async_copy(buf_a[last % 2], g_dst(last), sem_w[last % 2]).wait()


def _gather(a, b, sidx3, oidx3):
    mesh = plsc.VectorSubcoreMesh(core_axis_name="c", subcore_axis_name="s")
    f = pl.kernel(
        _gather_body,
        out_type=jax.ShapeDtypeStruct((T, D), jnp.float32),
        mesh=mesh,
        scratch_types=[
            pltpu.VMEM((GCHUNK, GK), jnp.int32),
            pltpu.VMEM((GCHUNK, GK), jnp.int32),
            pltpu.VMEM((GK, D), jnp.float32),
            pltpu.VMEM((GK, D), jnp.float32),
            pltpu.VMEM((GK, D), jnp.float32),
            pltpu.VMEM((GK, D), jnp.float32),
            pltpu.SemaphoreType.DMA,
            pltpu.SemaphoreType.DMA,
            pltpu.SemaphoreType.DMA,
            pltpu.SemaphoreType.DMA,
            pltpu.SemaphoreType.DMA,
            pltpu.SemaphoreType.DMA,
        ],
    )
    return f(a, b, sidx3, oidx3)


def _edge_body(g_ref, pred_ref, w1p_ref, b1_ref, w2m_ref, b2m_ref,
               h_ref, np_ref):
    p = jnp.dot(pred_ref[...], w1p_ref[...], preferred_element_type=jnp.float32)
    h = _lrelu(g_ref[...] + p + b1_ref[...])
    h_ref[...] = h
    mid = jnp.dot(h, w2m_ref[...], preferred_element_type=jnp.float32) + b2m_ref[...]
    np_ref[...] = _lrelu(mid)


def _edge_mlp(g, pred, w1p, b1, w2m, b2m):
    bt = 2000
    grid = (T // bt,)
    return pl.pallas_call(
        _edge_body,
        grid=grid,
        in_specs=[
            pl.BlockSpec((bt, D), lambda i: (i, 0)),
            pl.BlockSpec((bt, D), lambda i: (i, 0)),
            pl.BlockSpec((D, D), lambda i: (0, 0)),
            pl.BlockSpec((1, D), lambda i: (0, 0)),
            pl.BlockSpec((D, D), lambda i: (0, 0)),
            pl.BlockSpec((1, D), lambda i: (0, 0)),
        ],
        out_specs=[
            pl.BlockSpec((bt, D), lambda i: (i, 0)),
            pl.BlockSpec((bt, D), lambda i: (i, 0)),
        ],
        out_shape=[
            jax.ShapeDtypeStruct((T, D), jnp.float32),
            jax.ShapeDtypeStruct((T, D), jnp.float32),
        ],
    )(g, pred, w1p, b1, w2m, b2m)


def _scatter_body(h_hbm, sidx_hbm, oidx_hbm, hs_hbm, ho_hbm,
                  acc, hbuf0, hbuf1, idxbuf0, idxbuf1,
                  sem_h0, sem_h1, sem_i0, sem_i1, sem_s0, sem_s1):
    cid = lax.axis_index("c")
    sid = lax.axis_index("s")
    hbuf = (hbuf0, hbuf1)
    idxbuf = (idxbuf0, idxbuf1)
    sem_h = (sem_h0, sem_h1)
    sem_i = (sem_i0, sem_i1)
    sem_s = (sem_s0, sem_s1)

    @pl.loop(0, SK)
    def _fill(r):
        for c in range(D // 16):
            hbuf0[r, pl.ds(c * 16, 16)] = jnp.zeros((16,), jnp.float32)

    @pl.loop(0, OSLICE // SK)
    def _zero(k):
        pltpu.sync_copy(hbuf0, acc.at[pl.ds(sid * OSLICE + k * SK, SK)])

    plsc.subcore_barrier()

    extra = jnp.where(sid < SREM, 1, 0)
    base_j = sid * SPS + jnp.minimum(sid, SREM)
    n = SPS + extra

    def run(idx_hbm):
        def issue(i, s):
            j = base_j + i
            pltpu.async_copy(idx_hbm.at[pl.ds(j * SK, SK)], idxbuf[s], sem_i[s])
            pltpu.async_copy(h_hbm.at[pl.ds(j * SK, SK)], hbuf[s], sem_h[s])

        def wait(i, s):
            j = base_j + i
            pltpu.make_async_copy(
                idx_hbm.at[pl.ds(j * SK, SK)], idxbuf[s], sem_i[s]).wait()
            pltpu.make_async_copy(
                h_hbm.at[pl.ds(j * SK, SK)], hbuf[s], sem_h[s]).wait()

        issue(0, 0)

        def process(i, s):
            @pl.when(i < n)
            def _():
                wait(i, s)

                @pl.when(i >= 1)
                def _():
                    pltpu.make_async_copy(
                        hbuf[1 - s], acc.at[idxbuf[1 - s]], sem_s[1 - s]).wait()

                @pl.when(i + 1 < n)
                def _():
                    issue(i + 1, 1 - s)

                pltpu.async_copy(hbuf[s], acc.at[idxbuf[s]], sem_s[s], add=True)

        @pl.loop(0, (SPS + 2) // 2)
        def _outer(k):
            process(2 * k, 0)
            process(2 * k + 1, 1)

        @pl.when(n % 2 == 1)
        def _():
            pltpu.make_async_copy(hbuf[0], acc.at[idxbuf[0]], sem_s[0]).wait()

        @pl.when(n % 2 == 0)
        def _():
            pltpu.make_async_copy(hbuf[1], acc.at[idxbuf[1]], sem_s[1]).wait()

    @pl.when(cid == 0)
    def _():
        run(sidx_hbm)

    @pl.when(cid == 1)
    def _():
        run(oidx_hbm)

    plsc.subcore_barrier()

    sl = pl.ds(sid * OSLICE, OSLICE)

    @pl.when(cid == 0)
    def _():
        pltpu.sync_copy(acc.at[sl], hs_hbm.at[sl])

    @pl.when(cid == 1)
    def _():
        pltpu.sync_copy(acc.at[sl], ho_hbm.at[sl])


def _scatter(h, sidx, oidx):
    mesh = plsc.VectorSubcoreMesh(core_axis_name="c", subcore_axis_name="s")
    f = pl.kernel(
        _scatter_body,
        out_type=[
            jax.ShapeDtypeStruct((OP, D), jnp.float32),
            jax.ShapeDtypeStruct((OP, D), jnp.float32),
        ],
        mesh=mesh,
        scratch_types=[
            pltpu.MemorySpace.VMEM_SHARED((OP, D), jnp.float32),
            pltpu.VMEM((SK, D), jnp.float32),
            pltpu.VMEM((SK, D), jnp.float32),
            pltpu.VMEM((SK,), jnp.int32),
            pltpu.VMEM((SK,), jnp.int32),
            pltpu.SemaphoreType.DMA,
            pltpu.SemaphoreType.DMA,
            pltpu.SemaphoreType.DMA,
            pltpu.SemaphoreType.DMA,
            pltpu.SemaphoreType.DMA,
            pltpu.SemaphoreType.DMA,
        ],
    )
    return f(h, sidx, oidx)


def _count_body(sidx_hbm, oidx_hbm, cs_hbm, co_hbm,
                cacc, onesbuf, idxbuf0, idxbuf1,
                sem_i0, sem_i1, sem_s0, sem_s1):
    cid = lax.axis_index("c")
    sid = lax.axis_index("s")
    idxbuf = (idxbuf0, idxbuf1)
    sem_i = (sem_i0, sem_i1)
    sem_s = (sem_s0, sem_s1)

    @pl.loop(0, CK)
    def _fillz(r):
        for c in range(D // 16):
            onesbuf[r, pl.ds(c * 16, 16)] = jnp.zeros((16,), jnp.float32)

    @pl.loop(0, OSLICE // CK)
    def _zero(k):
        pltpu.sync_copy(onesbuf, cacc.at[pl.ds(sid * OSLICE + k * CK, CK)])

    @pl.loop(0, CK)
    def _fill1(r):
        for c in range(D // 16):
            onesbuf[r, pl.ds(c * 16, 16)] = jnp.ones((16,), jnp.float32)

    plsc.subcore_barrier()

    def run(idx_hbm):
        def issue(i, s):
            j = i * NS + sid
            pltpu.async_copy(idx_hbm.at[pl.ds(j * CK, CK)], idxbuf[s], sem_i[s])

        def wait_idx(i, s):
            j = i * NS + sid
            pltpu.make_async_copy(
                idx_hbm.at[pl.ds(j * CK, CK)], idxbuf[s], sem_i[s]).wait()

        issue(0, 0)

        def process(i, s):
            @pl.when(i * NS + sid < CCHUNK)
            def _():
                wait_idx(i, s)

                @pl.when(i >= 1)
                def _():
                    pltpu.make_async_copy(
                        onesbuf, cacc.at[idxbuf[1 - s]], sem_s[1 - s]).wait()

                @pl.when((i + 1) * NS + sid < CCHUNK)
                def _():
                    issue(i + 1, 1 - s)

                pltpu.async_copy(onesbuf, cacc.at[idxbuf[s]], sem_s[s], add=True)

        @pl.loop(0, ((CCHUNK + NS - 1) // NS + 2) // 2)
        def _outer(k):
            process(2 * k, 0)
            process(2 * k + 1, 1)

        ncb = jnp.where(sid < CCHUNK % NS, CCHUNK // NS + 1, CCHUNK // NS)

        @pl.when(ncb % 2 == 1)
        def _():
            pltpu.make_async_copy(onesbuf, cacc.at[idxbuf[0]], sem_s[0]).wait()

        @pl.when(ncb % 2 == 0)
        def _():
            pltpu.make_async_copy(onesbuf, cacc.at[idxbuf[1]], sem_s[1]).wait()

    @pl.when(cid == 0)
    def _():
        run(sidx_hbm)

    @pl.when(cid == 1)
    def _():
        run(oidx_hbm)

    plsc.subcore_barrier()

    sl = pl.ds(sid * OSLICE, OSLICE)

    @pl.when(cid == 0)
    def _():
        pltpu.sync_copy(cacc.at[sl], cs_hbm.at[sl])

    @pl.when(cid == 1)
    def _():
        pltpu.sync_copy(cacc.at[sl], co_hbm.at[sl])


def _counts(sidx, oidx):
    mesh = plsc.VectorSubcoreMesh(core_axis_name="c", subcore_axis_name="s")
    f = pl.kernel(
        _count_body,
        out_type=[
            jax.ShapeDtypeStruct((OP, D), jnp.float32),
            jax.ShapeDtypeStruct((OP, D), jnp.float32),
        ],
        mesh=mesh,
        scratch_types=[
            pltpu.MemorySpace.VMEM_SHARED((OP, D), jnp.float32),
            pltpu.VMEM((CK, D), jnp.float32),
            pltpu.VMEM((CK,), jnp.int32),
            pltpu.VMEM((CK,), jnp.int32),
            pltpu.SemaphoreType.DMA,
            pltpu.SemaphoreType.DMA,
            pltpu.SemaphoreType.DMA,
            pltpu.SemaphoreType.DMA,
        ],
    )
    return f(sidx, oidx)


def _node_body(hs_ref, ho_ref, cs_ref, co_ref,
               w2s_ref, w2o_ref,
               b2s_ref, b2o_ref, n2w1_ref, n2b1_ref, n2w2_ref, n2b2_ref,
               out_ref):
    cnt_s = cs_ref[:, 0:1]
    cnt_o = co_ref[:, 0:1]
    pooled = (
        jnp.dot(hs_ref[...], w2s_ref[...], preferred_element_type=jnp.float32)
        + jnp.dot(ho_ref[...], w2o_ref[...], preferred_element_type=jnp.float32)
        + cnt_s * b2s_ref[...]
        + cnt_o * b2o_ref[...]
    )
    pooled = pooled / jnp.maximum(cnt_s + cnt_o, 1.0)
    x = _lrelu(pooled)
    x = _lrelu(jnp.dot(x, n2w1_ref[...], preferred_element_type=jnp.float32)
               + n2b1_ref[...])
    out_ref[...] = _lrelu(jnp.dot(x, n2w2_ref[...],
                                  preferred_element_type=jnp.float32)
                          + n2b2_ref[...])


def _node_mlp(hs, ho, cs, co, w2s, w2o, b2s, b2o,
              n2w1, n2b1, n2w2, n2b2):
    bo = 1024
    grid = (OP // bo,)
    row = lambda i: (i, 0)
    fix = lambda i: (0, 0)
    return pl.pallas_call(
        _node_body,
        grid=grid,
        in_specs=[
            pl.BlockSpec((bo, D), row),
            pl.BlockSpec((bo, D), row),
            pl.BlockSpec((bo, D), row),
            pl.BlockSpec((bo, D), row),
            pl.BlockSpec((D, D), fix),
            pl.BlockSpec((D, D), fix),
            pl.BlockSpec((1, D), fix),
            pl.BlockSpec((1, D), fix),
            pl.BlockSpec((D, D), fix),
            pl.BlockSpec((1, D), fix),
            pl.BlockSpec((D, D), fix),
            pl.BlockSpec((1, D), fix),
        ],
        out_specs=pl.BlockSpec((bo, D), row),
        out_shape=jax.ShapeDtypeStruct((OP, D), jnp.float32),
    )(hs, ho, cs, co, w2s, w2o, b2s, b2o, n2w1, n2b1, n2w2, n2b2)


def kernel(obj_vecs, pred_vecs, edges,
           net1_W1, net1_b1, net1_W2, net1_b2,
           net2_W1, net2_b1, net2_W2, net2_b2):
    s_idx = edges[:, 0]
    o_idx = edges[:, 1]

    w1s = net1_W1[:D]
    w1p = net1_W1[D:2 * D]
    w1o = net1_W1[2 * D:]
    b1 = net1_b1.reshape(1, D)
    w2s = net1_W2[:, :D]
    w2m = net1_W2[:, D:2 * D]
    w2o = net1_W2[:, 2 * D:]
    b2s = net1_b2[:D].reshape(1, D)
    b2m = net1_b2[D:2 * D].reshape(1, D)
    b2o = net1_b2[2 * D:].reshape(1, D)

    nw = NC * NS
    sidx3 = s_idx.reshape(nw, GCHUNK, GK)
    oidx3 = o_idx.reshape(nw, GCHUNK, GK)

    a, b = _premul(obj_vecs, w1s, w1o)
    g = _gather(a, b, sidx3, oidx3)
    h, new_p = _edge_mlp(g, pred_vecs, w1p, b1, w2m, b2m)
    cs, co = _counts(s_idx, o_idx)
    hs, ho = _scatter(h, s_idx, o_idx)
    new_obj = _node_mlp(hs, ho, cs, co, w2s, w2o, b2s, b2o,
                        net2_W1, net2_b1.reshape(1, D),
                        net2_W2, net2_b2.reshape(1, D))
    return new_obj[:O], new_p


# speedup vs baseline: 1.0013x; 1.0013x over previous
import functools

import jax
import jax.numpy as jnp
from jax import lax
from jax.experimental import pallas as pl
from jax.experimental.pallas import tpu as pltpu
from jax.experimental.pallas import tpu_sc as plsc

O = 10000
T = 320000
D = 128
NEG = 0.01

NC = 2
NS = 16
GK = 80
EPW = T // (NC * NS)
GCHUNK = EPW // GK
SK = 64
NCHUNK = T // SK
SPS = NCHUNK // NS
SREM = NCHUNK - SPS * NS
CK = 128
CCHUNK = T // CK
OP = 10240
OSLICE = OP // NS


def _lrelu(x):
    return jnp.where(x >= 0, x, NEG * x)


def _premul_body(obj_ref, w1s_ref, w1o_ref, a_ref, b_ref):
    blk = obj_ref[...]
    a_ref[...] = jnp.dot(blk, w1s_ref[...], preferred_element_type=jnp.float32)
    b_ref[...] = jnp.dot(blk, w1o_ref[...], preferred_element_type=jnp.float32)


def _premul(obj_vecs, w1s, w1o):
    bo = 1000
    grid = (O // bo,)
    return pl.pallas_call(
        _premul_body,
        grid=grid,
        in_specs=[
            pl.BlockSpec((bo, D), lambda i: (i, 0)),
            pl.BlockSpec((D, D), lambda i: (0, 0)),
            pl.BlockSpec((D, D), lambda i: (0, 0)),
        ],
        out_specs=[
            pl.BlockSpec((bo, D), lambda i: (i, 0)),
            pl.BlockSpec((bo, D), lambda i: (i, 0)),
        ],
        out_shape=[
            jax.ShapeDtypeStruct((O, D), jnp.float32),
            jax.ShapeDtypeStruct((O, D), jnp.float32),
        ],
    )(obj_vecs, w1s, w1o)


def _gather_body(a_hbm, b_hbm, sidx_hbm, oidx_hbm, g_hbm,
                 sidx_v, oidx_v,
                 buf_a0, buf_a1, buf_b0, buf_b1,
                 sem_a0, sem_a1, sem_b0, sem_b1, sem_w0, sem_w1):
    cid = lax.axis_index("c")
    sid = lax.axis_index("s")
    wid = sid * NC + cid
    base = wid * EPW
    buf_a = (buf_a0, buf_a1)
    buf_b = (buf_b0, buf_b1)
    sem_a = (sem_a0, sem_a1)
    sem_b = (sem_b0, sem_b1)
    sem_w = (sem_w0, sem_w1)
    pltpu.sync_copy(sidx_hbm.at[wid], sidx_v)
    pltpu.sync_copy(oidx_hbm.at[wid], oidx_v)

    def issue(i, s):
        pltpu.async_copy(a_hbm.at[sidx_v.at[i]], buf_a[s], sem_a[s])
        pltpu.async_copy(b_hbm.at[oidx_v.at[i]], buf_b[s], sem_b[s])

    def g_dst(i):
        return g_hbm.at[pl.ds(base + i * GK, GK)]

    issue(0, 0)

    def process(i, s):
        @pl.when(i < GCHUNK)
        def _():
            pltpu.make_async_copy(a_hbm.at[sidx_v.at[i]], buf_a[s], sem_a[s]).wait()
            pltpu.make_async_copy(b_hbm.at[oidx_v.at[i]], buf_b[s], sem_b[s]).wait()

            @pl.when(i >= 1)
            def _():
                pltpu.make_async_copy(buf_a[1 - s], g_dst(i - 1), sem_w[1 - s]).wait()

            @pl.when(i + 1 < GCHUNK)
            def _():
                issue(i + 1, 1 - s)

            @pl.loop(0, GK)
            def _row(r):
                for c in range(D // 16):
                    sl = pl.ds(c * 16, 16)
                    buf_a[s][r, sl] = buf_a[s][r, sl] + buf_b[s][r, sl]

            pltpu.async_copy(buf_a[s], g_dst(i), sem_w[s])

    @pl.loop(0, (GCHUNK + 1) // 2)
    def _outer(k):
        process(2 * k, 0)
        process(2 * k + 1, 1)

    last = GCHUNK - 1
    pltpu.make_async_copy(buf_a[last % 2], g_dst(last), sem_w[last % 2]).wait()


def _gather(a, b, sidx3, oidx3):
    mesh = plsc.VectorSubcoreMesh(core_axis_name="c", subcore_axis_name="s")
    f = pl.kernel(
        _gather_body,
        out_type=jax.ShapeDtypeStruct((T, D), jnp.float32),
        mesh=mesh,
        scratch_types=[
            pltpu.VMEM((GCHUNK, GK), jnp.int32),
            pltpu.VMEM((GCHUNK, GK), jnp.int32),
            pltpu.VMEM((GK, D), jnp.float32),
            pltpu.VMEM((GK, D), jnp.float32),
            pltpu.VMEM((GK, D), jnp.float32),
            pltpu.VMEM((GK, D), jnp.float32),
            pltpu.SemaphoreType.DMA,
            pltpu.SemaphoreType.DMA,
            pltpu.SemaphoreType.DMA,
            pltpu.SemaphoreType.DMA,
            pltpu.SemaphoreType.DMA,
            pltpu.SemaphoreType.DMA,
        ],
    )
    return f(a, b, sidx3, oidx3)


def _edge_body(g_ref, pred_ref, w1p_ref, b1_ref, w2m_ref, b2m_ref,
               h_ref, np_ref):
    p = jnp.dot(pred_ref[...], w1p_ref[...], preferred_element_type=jnp.float32)
    h = _lrelu(g_ref[...] + p + b1_ref[...])
    h_ref[...] = h
    mid = jnp.dot(h, w2m_ref[...], preferred_element_type=jnp.float32) + b2m_ref[...]
    np_ref[...] = _lrelu(mid)


def _edge_mlp(g, pred, w1p, b1, w2m, b2m):
    bt = 2000
    grid = (T // bt,)
    return pl.pallas_call(
        _edge_body,
        grid=grid,
        in_specs=[
            pl.BlockSpec((bt, D), lambda i: (i, 0)),
            pl.BlockSpec((bt, D), lambda i: (i, 0)),
            pl.BlockSpec((D, D), lambda i: (0, 0)),
            pl.BlockSpec((1, D), lambda i: (0, 0)),
            pl.BlockSpec((D, D), lambda i: (0, 0)),
            pl.BlockSpec((1, D), lambda i: (0, 0)),
        ],
        out_specs=[
            pl.BlockSpec((bt, D), lambda i: (i, 0)),
            pl.BlockSpec((bt, D), lambda i: (i, 0)),
        ],
        out_shape=[
            jax.ShapeDtypeStruct((T, D), jnp.float32),
            jax.ShapeDtypeStruct((T, D), jnp.float32),
        ],
    )(g, pred, w1p, b1, w2m, b2m)


def _scatter_body(h_hbm, sidx_hbm, oidx_hbm, hs_hbm, ho_hbm,
                  acc, hbuf0, hbuf1, idxbuf0, idxbuf1,
                  sem_h0, sem_h1, sem_i0, sem_i1):
    cid = lax.axis_index("c")
    sid = lax.axis_index("s")
    hbuf = (hbuf0, hbuf1)
    idxbuf = (idxbuf0, idxbuf1)
    sem_h = (sem_h0, sem_h1)
    sem_i = (sem_i0, sem_i1)

    @pl.loop(0, SK)
    def _fill(r):
        for c in range(D // 16):
            hbuf0[r, pl.ds(c * 16, 16)] = jnp.zeros((16,), jnp.float32)

    @pl.loop(0, OSLICE // SK)
    def _zero(k):
        pltpu.sync_copy(hbuf0, acc.at[pl.ds(sid * OSLICE + k * SK, SK)])

    plsc.subcore_barrier()

    extra = jnp.where(sid < SREM, 1, 0)
    base_j = sid * SPS + jnp.minimum(sid, SREM)
    n = SPS + extra

    def run(idx_hbm):
        def issue(i, s):
            j = base_j + i
            pltpu.async_copy(idx_hbm.at[pl.ds(j * SK, SK)], idxbuf[s], sem_i[s])
            pltpu.async_copy(h_hbm.at[pl.ds(j * SK, SK)], hbuf[s], sem_h[s])

        def wait(i, s):
            j = base_j + i
            pltpu.make_async_copy(
                idx_hbm.at[pl.ds(j * SK, SK)], idxbuf[s], sem_i[s]).wait()
            pltpu.make_async_copy(
                h_hbm.at[pl.ds(j * SK, SK)], hbuf[s], sem_h[s]).wait()

        issue(0, 0)

        def process(i, s):
            @pl.when(i < n)
            def _():
                wait(i, s)

                @pl.when(i + 1 < n)
                def _():
                    issue(i + 1, 1 - s)

                pltpu.sync_copy(hbuf[s], acc.at[idxbuf[s]], add=True)

        @pl.loop(0, (SPS + 2) // 2)
        def _outer(k):
            process(2 * k, 0)
            process(2 * k + 1, 1)

    @pl.when(cid == 0)
    def _():
        run(sidx_hbm)

    @pl.when(cid == 1)
    def _():
        run(oidx_hbm)

    plsc.subcore_barrier()

    sl = pl.ds(sid * OSLICE, OSLICE)

    @pl.when(cid == 0)
    def _():
        pltpu.sync_copy(acc.at[sl], hs_hbm.at[sl])

    @pl.when(cid == 1)
    def _():
        pltpu.sync_copy(acc.at[sl], ho_hbm.at[sl])


def _scatter(h, sidx, oidx):
    mesh = plsc.VectorSubcoreMesh(core_axis_name="c", subcore_axis_name="s")
    f = pl.kernel(
        _scatter_body,
        out_type=[
            jax.ShapeDtypeStruct((OP, D), jnp.float32),
            jax.ShapeDtypeStruct((OP, D), jnp.float32),
        ],
        mesh=mesh,
        scratch_types=[
            pltpu.MemorySpace.VMEM_SHARED((OP, D), jnp.float32),
            pltpu.VMEM((SK, D), jnp.float32),
            pltpu.VMEM((SK, D), jnp.float32),
            pltpu.VMEM((SK,), jnp.int32),
            pltpu.VMEM((SK,), jnp.int32),
            pltpu.SemaphoreType.DMA,
            pltpu.SemaphoreType.DMA,
            pltpu.SemaphoreType.DMA,
            pltpu.SemaphoreType.DMA,
        ],
    )
    return f(h, sidx, oidx)


def _count_body(sidx_hbm, oidx_hbm, cs_hbm, co_hbm,
                cacc, onesbuf, idxbuf):
    cid = lax.axis_index("c")
    sid = lax.axis_index("s")

    @pl.loop(0, CK)
    def _fillz(r):
        for c in range(D // 16):
            onesbuf[r, pl.ds(c * 16, 16)] = jnp.zeros((16,), jnp.float32)

    @pl.loop(0, OSLICE // CK)
    def _zero(k):
        pltpu.sync_copy(onesbuf, cacc.at[pl.ds(sid * OSLICE + k * CK, CK)])

    @pl.loop(0, CK)
    def _fill1(r):
        for c in range(D // 16):
            onesbuf[r, pl.ds(c * 16, 16)] = jnp.ones((16,), jnp.float32)

    plsc.subcore_barrier()

    def run(idx_hbm):
        @pl.loop(0, (CCHUNK + NS - 1) // NS)
        def _chunk(i):
            j = i * NS + sid

            @pl.when(j < CCHUNK)
            def _():
                pltpu.sync_copy(idx_hbm.at[pl.ds(j * CK, CK)], idxbuf)
                pltpu.sync_copy(onesbuf, cacc.at[idxbuf], add=True)

    @pl.when(cid == 0)
    def _():
        run(sidx_hbm)

    @pl.when(cid == 1)
    def _():
        run(oidx_hbm)

    plsc.subcore_barrier()

    sl = pl.ds(sid * OSLICE, OSLICE)

    @pl.when(cid == 0)
    def _():
        pltpu.sync_copy(cacc.at[sl], cs_hbm.at[sl])

    @pl.when(cid == 1)
    def _():
        pltpu.sync_copy(cacc.at[sl], co_hbm.at[sl])


def _counts(sidx, oidx):
    mesh = plsc.VectorSubcoreMesh(core_axis_name="c", subcore_axis_name="s")
    f = pl.kernel(
        _count_body,
        out_type=[
            jax.ShapeDtypeStruct((OP, D), jnp.float32),
            jax.ShapeDtypeStruct((OP, D), jnp.float32),
        ],
        mesh=mesh,
        scratch_types=[
            pltpu.MemorySpace.VMEM_SHARED((OP, D), jnp.float32),
            pltpu.VMEM((CK, D), jnp.float32),
            pltpu.VMEM((CK,), jnp.int32),
        ],
    )
    return f(sidx, oidx)


def _node_body(hs_ref, ho_ref, cs_ref, co_ref,
               w2s_ref, w2o_ref,
               b2s_ref, b2o_ref, n2w1_ref, n2b1_ref, n2w2_ref, n2b2_ref,
               out_ref):
    cnt_s = cs_ref[:, 0:1]
    cnt_o = co_ref[:, 0:1]
    pooled = (
        jnp.dot(hs_ref[...], w2s_ref[...], preferred_element_type=jnp.float32)
        + jnp.dot(ho_ref[...], w2o_ref[...], preferred_element_type=jnp.float32)
        + cnt_s * b2s_ref[...]
        + cnt_o * b2o_ref[...]
    )
    pooled = pooled / jnp.maximum(cnt_s + cnt_o, 1.0)
    x = _lrelu(pooled)
    x = _lrelu(jnp.dot(x, n2w1_ref[...], preferred_element_type=jnp.float32)
               + n2b1_ref[...])
    out_ref[...] = _lrelu(jnp.dot(x, n2w2_ref[...],
                                  preferred_element_type=jnp.float32)
                          + n2b2_ref[...])


def _node_mlp(hs, ho, cs, co, w2s, w2o, b2s, b2o,
              n2w1, n2b1, n2w2, n2b2):
    bo = 1024
    grid = (OP // bo,)
    row = lambda i: (i, 0)
    fix = lambda i: (0, 0)
    return pl.pallas_call(
        _node_body,
        grid=grid,
        in_specs=[
            pl.BlockSpec((bo, D), row),
            pl.BlockSpec((bo, D), row),
            pl.BlockSpec((bo, D), row),
            pl.BlockSpec((bo, D), row),
            pl.BlockSpec((D, D), fix),
            pl.BlockSpec((D, D), fix),
            pl.BlockSpec((1, D), fix),
            pl.BlockSpec((1, D), fix),
            pl.BlockSpec((D, D), fix),
            pl.BlockSpec((1, D), fix),
            pl.BlockSpec((D, D), fix),
            pl.BlockSpec((1, D), fix),
        ],
        out_specs=pl.BlockSpec((bo, D), row),
        out_shape=jax.ShapeDtypeStruct((OP, D), jnp.float32),
    )(hs, ho, cs, co, w2s, w2o, b2s, b2o, n2w1, n2b1, n2w2, n2b2)


def kernel(obj_vecs, pred_vecs, edges,
           net1_W1, net1_b1, net1_W2, net1_b2,
           net2_W1, net2_b1, net2_W2, net2_b2):
    s_idx = edges[:, 0]
    o_idx = edges[:, 1]

    w1s = net1_W1[:D]
    w1p = net1_W1[D:2 * D]
    w1o = net1_W1[2 * D:]
    b1 = net1_b1.reshape(1, D)
    w2s = net1_W2[:, :D]
    w2m = net1_W2[:, D:2 * D]
    w2o = net1_W2[:, 2 * D:]
    b2s = net1_b2[:D].reshape(1, D)
    b2m = net1_b2[D:2 * D].reshape(1, D)
    b2o = net1_b2[2 * D:].reshape(1, D)

    nw = NC * NS
    sidx3 = s_idx.reshape(nw, GCHUNK, GK)
    oidx3 = o_idx.reshape(nw, GCHUNK, GK)

    a, b = _premul(obj_vecs, w1s, w1o)
    g = _gather(a, b, sidx3, oidx3)
    cs, co = _counts(s_idx, o_idx)
    h, new_p = _edge_mlp(g, pred_vecs, w1p, b1, w2m, b2m)
    hs, ho = _scatter(h, s_idx, o_idx)
    new_obj = _node_mlp(hs, ho, cs, co, w2s, w2o, b2s, b2o,
                        net2_W1, net2_b1.reshape(1, D),
                        net2_W2, net2_b2.reshape(1, D))
    return new_obj[:O], new_p
